# Initial kernel scaffold; baseline (speedup 1.0000x reference)
import jax
import jax.numpy as jnp
from jax.experimental import pallas as pl


def kernel(feature, segments):
    raise NotImplementedError("write your pallas kernel here")


# speedup vs baseline: 281.6621x; 281.6621x over previous
import functools

import jax
import jax.numpy as jnp
from jax.experimental import pallas as pl
from jax.experimental.pallas import tpu as pltpu

_T = 128
_LEVELS = 7
_N = 512
_C = 512
_B = 8


def _bounds_and_onehots(seg_ref):
    a = jnp.clip(seg_ref[...], 0.0, 125.0)
    s0 = jnp.floor(a[0:1, :])
    s1 = jnp.ceil(a[1:2, :])
    s1 = jnp.where(s0 == s1, jnp.ceil(a[1:2, :] + 1.0), s1)
    e0 = jnp.floor(a[2:3, :])
    e1 = jnp.ceil(a[3:4, :])
    e0 = jnp.where(e0 == e1, jnp.floor(a[2:3, :] - 1.0), e0)

    def idx_pair(lo_f, hi_f):
        lo = jnp.maximum(lo_f, 0.0).astype(jnp.int32)
        hi = hi_f.astype(jnp.int32)
        w = hi - lo
        k = ((w >= 2).astype(jnp.int32) + (w >= 4).astype(jnp.int32)
             + (w >= 8).astype(jnp.int32) + (w >= 16).astype(jnp.int32)
             + (w >= 32).astype(jnp.int32) + (w >= 64).astype(jnp.int32))
        two_k = jnp.left_shift(jnp.int32(1), k)
        j1 = k * _T + lo
        j2 = k * _T + hi - two_k
        empty = w < 1
        return j1, j2, empty

    j1s, j2s, empty_s = idx_pair(s0, s1)
    j1e, j2e, empty_e = idx_pair(e0, e1)

    iota = jax.lax.broadcasted_iota(jnp.int32, (_LEVELS * _T, _N), 0)
    onehot = lambda j: (iota == j).astype(jnp.float32)
    return (onehot(j1s), onehot(j2s), empty_s,
            onehot(j1e), onehot(j2e), empty_e)


def _body(f_ref, seg_ref, out_ref):
    es1, es2, empty_s, ee1, ee2, empty_e = _bounds_and_onehots(seg_ref)

    p = f_ref[0]
    tables = [p]
    for s in (1, 2, 4, 8, 16, 32):
        shifted = jnp.concatenate([p[:, s:], p[:, :s]], axis=-1)
        p = jnp.maximum(p, shifted)
        tables.append(p)
    table = jnp.concatenate(tables, axis=-1)

    neg_inf = jnp.float32(-jnp.inf)

    def half(tab_half, e1m, e2m, empty):
        g1 = jnp.dot(tab_half, e1m, preferred_element_type=jnp.float32)
        g2 = jnp.dot(tab_half, e2m, preferred_element_type=jnp.float32)
        out = jnp.maximum(g1, g2)
        return jnp.where(empty, neg_inf, out)

    out_ref[0, : _C // 2, :] = half(table[: _C // 2], es1, es2, empty_s)
    out_ref[0, _C // 2 :, :] = half(table[_C // 2 :], ee1, ee2, empty_e)


@jax.jit
def _run(feature, seg_t):
    return pl.pallas_call(
        _body,
        grid=(_B,),
        in_specs=[
            pl.BlockSpec((1, _C, _T), lambda b: (b, 0, 0)),
            pl.BlockSpec((4, _N), lambda b: (0, 0)),
        ],
        out_specs=pl.BlockSpec((1, _C, _N), lambda b: (b, 0, 0)),
        out_shape=jax.ShapeDtypeStruct((_B, _C, _N), jnp.float32),
    )(feature, seg_t)


def kernel(feature, segments):
    seg_t = segments[0].T
    return _run(feature, seg_t)
